# baseline (speedup 1.0000x reference)
import functools
import math

import jax
import jax.numpy as jnp
from jax.experimental import pallas as pl
from jax.experimental.pallas import tpu as pltpu
from jax.experimental.pallas import tpu_sc as plsc

_D = 128
_SCALE = math.sqrt(float(_D))
_BW = 8
_CHUNKS = 4


def _sc_gather(table, idx, batch, seq):
    mesh = plsc.VectorSubcoreMesh(core_axis_name="c", subcore_axis_name="s")

    @functools.partial(
        pl.kernel,
        out_type=jax.ShapeDtypeStruct((batch, seq, _D), jnp.float32),
        mesh=mesh,
        scratch_types=[pltpu.SemaphoreType.DMA],
    )
    def k(t_hbm, i_hbm, o_hbm, sem):
        def body(i_vmem, o_vmem):
            copies = [
                pltpu.async_copy(t_hbm.at[i_vmem.at[j]], o_vmem.at[j], sem)
                for j in range(_BW)
            ]
            for c in copies:
                c.wait()

        pltpu.emit_pipeline(
            body,
            grid=(batch // _BW,),
            in_specs=[pl.BlockSpec((_BW, seq), index_map=lambda i: (i, 0))],
            out_specs=[pl.BlockSpec((_BW, seq, _D), index_map=lambda i: (i, 0, 0))],
            core_axis_name=("c", "s"),
            dimension_semantics=(pltpu.PARALLEL,),
        )(i_hbm, o_hbm)

    return k(table, idx)


def kernel(inputs, table):
    batch, seq = inputs.shape
    cb = batch // _CHUNKS
    scale = jnp.float32(_SCALE)
    parts = [
        _sc_gather(table, inputs[c * cb:(c + 1) * cb], cb, seq) * scale
        for c in range(_CHUNKS)
    ]
    return jnp.concatenate(parts, axis=0)


# speedup vs baseline: 1.5328x; 1.5328x over previous
import functools
import math

import jax
import jax.numpy as jnp
from jax.experimental import pallas as pl
from jax.experimental.pallas import tpu as pltpu
from jax.experimental.pallas import tpu_sc as plsc

_D = 128
_SCALE = math.sqrt(float(_D))
_BW = 8


def _scale_block(t_ref, o_ref):
    o_ref[...] = t_ref[...] * _SCALE


def _scaled_table(table):
    rows, d = table.shape
    blk = 1000
    return pl.pallas_call(
        _scale_block,
        out_shape=jax.ShapeDtypeStruct(table.shape, table.dtype),
        grid=(rows // blk,),
        in_specs=[pl.BlockSpec((blk, d), lambda i: (i, 0))],
        out_specs=pl.BlockSpec((blk, d), lambda i: (i, 0)),
        compiler_params=pltpu.CompilerParams(
            dimension_semantics=("parallel",),
        ),
    )(table)


def _sc_gather(table_scaled, idx, batch, seq):
    mesh = plsc.VectorSubcoreMesh(core_axis_name="c", subcore_axis_name="s")

    @functools.partial(
        pl.kernel,
        out_type=jax.ShapeDtypeStruct((batch, seq, _D), jnp.float32),
        mesh=mesh,
        scratch_types=[pltpu.SemaphoreType.DMA],
    )
    def k(t_hbm, i_hbm, o_hbm, sem):
        def body(i_vmem, o_vmem):
            copies = [
                pltpu.async_copy(t_hbm.at[i_vmem.at[j]], o_vmem.at[j], sem)
                for j in range(_BW)
            ]
            for c in copies:
                c.wait()

        pltpu.emit_pipeline(
            body,
            grid=(batch // _BW,),
            in_specs=[pl.BlockSpec((_BW, seq), index_map=lambda i: (i, 0))],
            out_specs=[pl.BlockSpec((_BW, seq, _D), index_map=lambda i: (i, 0, 0))],
            core_axis_name=("c", "s"),
            dimension_semantics=(pltpu.PARALLEL,),
        )(i_hbm, o_hbm)

    return k(table_scaled, idx)


def kernel(inputs, table):
    batch, seq = inputs.shape
    ts = _scaled_table(table)
    return _sc_gather(ts, inputs, batch, seq)
